# baseline (speedup 1.0000x reference)
import functools

import jax
from jax import lax
import jax.numpy as jnp
from jax.experimental import pallas as pl
from jax.experimental.pallas import tpu as pltpu
from jax.experimental.pallas import tpu_sc as plsc

_B = 1024
_RANK = 32
_N = 100000
_NR = 1000
_BE = 512
_K = 4
_NP = 100352
_TAIL = _N - 48 * 2048 - 3 * 512
_PACK = 4
_NC = 2
_NS = 16
_NW = _NC * _NS
_BPW = _B // _NW


def _sc_gather_packed(e4, r4, lhs_row, rel_row, rhs_row):
    mesh = plsc.VectorSubcoreMesh(core_axis_name="c", subcore_axis_name="s")
    out = jax.ShapeDtypeStruct((_B, 128), jnp.float32)

    @functools.partial(
        pl.kernel, mesh=mesh, out_type=(out, out, out),
        compiler_params=pltpu.CompilerParams(use_tc_tiling_on_sc=False),
        scratch_types=[
            pltpu.VMEM((_BPW,), jnp.int32),
            pltpu.VMEM((_BPW, 128), jnp.float32),
            pltpu.VMEM((_BPW,), jnp.int32),
            pltpu.VMEM((_BPW, 128), jnp.float32),
            pltpu.VMEM((_BPW,), jnp.int32),
            pltpu.VMEM((_BPW, 128), jnp.float32),
            pltpu.SemaphoreType.DMA,
            pltpu.SemaphoreType.DMA,
            pltpu.SemaphoreType.DMA,
        ],
    )
    def k(e4_hbm, r4_hbm, li_hbm, ri_hbm, ti_hbm, lhs_hbm, rel_hbm,
          rhs_hbm, li_v, lrow_v, ri_v, rrow_v, ti_v, trow_v, sem1, sem2,
          sem3):
        wid = lax.axis_index("s") * _NC + lax.axis_index("c")
        base = wid * _BPW
        pltpu.sync_copy(li_hbm.at[pl.ds(base, _BPW)], li_v)
        pltpu.sync_copy(ri_hbm.at[pl.ds(base, _BPW)], ri_v)
        pltpu.sync_copy(ti_hbm.at[pl.ds(base, _BPW)], ti_v)
        cp1 = pltpu.async_copy(e4_hbm.at[li_v], lrow_v, sem1)
        cp2 = pltpu.async_copy(r4_hbm.at[ri_v], rrow_v, sem2)
        cp3 = pltpu.async_copy(e4_hbm.at[ti_v], trow_v, sem3)
        cp1.wait()
        pltpu.sync_copy(lrow_v, lhs_hbm.at[pl.ds(base, _BPW)])
        cp2.wait()
        pltpu.sync_copy(rrow_v, rel_hbm.at[pl.ds(base, _BPW)])
        cp3.wait()
        pltpu.sync_copy(trow_v, rhs_hbm.at[pl.ds(base, _BPW)])

    return k(e4, r4, lhs_row, rel_row, rhs_row)


def _select_chunk(packed, chunk):
    out = jnp.zeros((_B, _RANK), jnp.float32)
    for c in range(_PACK):
        piece = packed[:, c * _RANK:(c + 1) * _RANK]
        out = jnp.where(chunk == c, piece, out)
    return out


def _score_block_kernel(e_ref, lhs4_ref, rel4_ref, rhs4_ref, c0_ref,
                        c1_ref, c2_ref, out_hbm, lhs_out, rel_out,
                        rhs_out, buf, q16_buf, sems):
    i = pl.program_id(0)
    nsteps = pl.num_programs(0)

    @pl.when(i == 0)
    def _():
        lhs = _select_chunk(lhs4_ref[...], c0_ref[...])
        rel = _select_chunk(rel4_ref[...], c1_ref[...])
        rhs = _select_chunk(rhs4_ref[...], c2_ref[...])
        lhs_out[...] = lhs
        rel_out[...] = rel
        rhs_out[...] = rhs
        q16_buf[...] = (lhs * rel).astype(jnp.bfloat16)

    q16 = q16_buf[...]

    for j in range(_K):
        @pl.when(i > 0)
        def _(j=j):
            pltpu.make_async_copy(
                buf.at[j],
                out_hbm.at[pl.ds(((i - 1) * _K + j) * _BE, _BE), :],
                sems.at[j]).wait()

        acc = jax.lax.dot_general(
            e_ref[:, j * _BE:(j + 1) * _BE].astype(jnp.bfloat16), q16,
            dimension_numbers=(((0,), (1,)), ((), ())),
            preferred_element_type=jnp.float32,
        )
        buf[j] = acc
        if j < _K - 1:
            pltpu.make_async_copy(
                buf.at[j],
                out_hbm.at[pl.ds((i * _K + j) * _BE, _BE), :],
                sems.at[j]).start()
        else:
            @pl.when(i < nsteps - 1)
            def _():
                pltpu.make_async_copy(
                    buf.at[j],
                    out_hbm.at[pl.ds((i * _K + j) * _BE, _BE), :],
                    sems.at[j]).start()

            @pl.when(i == nsteps - 1)
            def _():
                pltpu.make_async_copy(
                    buf.at[j, pl.ds(0, _TAIL), :],
                    out_hbm.at[pl.ds((i * _K + j) * _BE, _TAIL), :],
                    sems.at[j]).start()

    @pl.when(i == nsteps - 1)
    def _():
        for j in range(_K - 1):
            pltpu.make_async_copy(
                buf.at[j],
                out_hbm.at[pl.ds((i * _K + j) * _BE, _BE), :],
                sems.at[j]).wait()
        pltpu.make_async_copy(
            buf.at[_K - 1, pl.ds(0, _TAIL), :],
            out_hbm.at[pl.ds((i * _K + _K - 1) * _BE, _TAIL), :],
            sems.at[_K - 1]).wait()


def _tc_scores_t(lhs4, rel4, rhs4, c0, c1, c2, entity_emb):
    e_pad = jnp.pad(entity_emb.T, ((0, 0), (0, _NP - _N)))
    grid = (_NP // (_K * _BE),)
    small = jax.ShapeDtypeStruct((_B, _RANK), jnp.float32)
    return pl.pallas_call(
        _score_block_kernel,
        grid=grid,
        in_specs=[
            pl.BlockSpec((_RANK, _K * _BE), lambda i: (0, i)),
            pl.BlockSpec((_B, 128), lambda i: (0, 0)),
            pl.BlockSpec((_B, 128), lambda i: (0, 0)),
            pl.BlockSpec((_B, 128), lambda i: (0, 0)),
            pl.BlockSpec((_B, 1), lambda i: (0, 0)),
            pl.BlockSpec((_B, 1), lambda i: (0, 0)),
            pl.BlockSpec((_B, 1), lambda i: (0, 0)),
        ],
        out_specs=[
            pl.BlockSpec(memory_space=pl.ANY),
            pl.BlockSpec((_B, _RANK), lambda i: (0, 0)),
            pl.BlockSpec((_B, _RANK), lambda i: (0, 0)),
            pl.BlockSpec((_B, _RANK), lambda i: (0, 0)),
        ],
        out_shape=[
            jax.ShapeDtypeStruct((_N, _B), jnp.float32),
            small, small, small,
        ],
        scratch_shapes=[
            pltpu.VMEM((_K, _BE, _B), jnp.float32),
            pltpu.VMEM((_B, _RANK), jnp.bfloat16),
            pltpu.SemaphoreType.DMA((_K,)),
        ],
        compiler_params=pltpu.CompilerParams(
            dimension_semantics=("arbitrary",),
            vmem_limit_bytes=64 * 1024 * 1024,
        ),
    )(e_pad, lhs4, rel4, rhs4, c0, c1, c2)


def kernel(queries, entity_emb, rel_emb):
    i0 = queries[:, 0]
    i1 = queries[:, 1]
    i2 = queries[:, 2]
    e4 = entity_emb.reshape(_N // _PACK, _RANK * _PACK)
    r4 = rel_emb.reshape(_NR // _PACK, _RANK * _PACK)
    lhs4, rel4, rhs4 = _sc_gather_packed(
        e4, r4, i0 // _PACK, i1 // _PACK, i2 // _PACK)
    c0 = (i0 % _PACK).reshape(_B, 1)
    c1 = (i1 % _PACK).reshape(_B, 1)
    c2 = (i2 % _PACK).reshape(_B, 1)
    scores_t, lhs, rel, rhs = _tc_scores_t(
        lhs4, rel4, rhs4, c0, c1, c2, entity_emb)
    return (scores_t.T, lhs, rel, rhs)


# speedup vs baseline: 1.0438x; 1.0438x over previous
import functools

import jax
from jax import lax
import jax.numpy as jnp
from jax.experimental import pallas as pl
from jax.experimental.pallas import tpu as pltpu
from jax.experimental.pallas import tpu_sc as plsc

_B = 1024
_RANK = 32
_N = 100000
_BN = 2048
_BE = 512
_K = 4
_NC = 2
_NS = 16
_NW = _NC * _NS
_BPW = _B // _NW


def _sc_gather_all(entity_emb, rel_emb, lhs_idx, rel_idx, rhs_idx):
    mesh = plsc.VectorSubcoreMesh(core_axis_name="c", subcore_axis_name="s")
    out = jax.ShapeDtypeStruct((_B, _RANK), jnp.float32)

    @functools.partial(
        pl.kernel, mesh=mesh, out_type=(out, out, out),
        compiler_params=pltpu.CompilerParams(use_tc_tiling_on_sc=False),
        scratch_types=[
            pltpu.VMEM((_BPW,), jnp.int32),
            pltpu.VMEM((_BPW, _RANK), jnp.float32),
            pltpu.VMEM((_BPW,), jnp.int32),
            pltpu.VMEM((_BPW, _RANK), jnp.float32),
            pltpu.VMEM((_BPW,), jnp.int32),
            pltpu.VMEM((_BPW, _RANK), jnp.float32),
            pltpu.SemaphoreType.DMA,
            pltpu.SemaphoreType.DMA,
            pltpu.SemaphoreType.DMA,
        ],
    )
    def k(ent_hbm, relt_hbm, li_hbm, ri_hbm, ti_hbm, lhs_hbm, rel_hbm,
          rhs_hbm, li_v, lrow_v, ri_v, rrow_v, ti_v, trow_v, sem1, sem2,
          sem3):
        wid = lax.axis_index("s") * _NC + lax.axis_index("c")
        base = wid * _BPW
        pltpu.sync_copy(li_hbm.at[pl.ds(base, _BPW)], li_v)
        pltpu.sync_copy(ri_hbm.at[pl.ds(base, _BPW)], ri_v)
        pltpu.sync_copy(ti_hbm.at[pl.ds(base, _BPW)], ti_v)
        cp1 = pltpu.async_copy(ent_hbm.at[li_v], lrow_v, sem1)
        cp2 = pltpu.async_copy(relt_hbm.at[ri_v], rrow_v, sem2)
        cp3 = pltpu.async_copy(ent_hbm.at[ti_v], trow_v, sem3)
        cp1.wait()
        pltpu.sync_copy(lrow_v, lhs_hbm.at[pl.ds(base, _BPW)])
        cp2.wait()
        pltpu.sync_copy(rrow_v, rel_hbm.at[pl.ds(base, _BPW)])
        cp3.wait()
        pltpu.sync_copy(trow_v, rhs_hbm.at[pl.ds(base, _BPW)])

    return k(entity_emb, rel_emb, lhs_idx, rel_idx, rhs_idx)


_NP = 100352
_TAIL = _N - 48 * 2048 - 3 * 512


def _score_block_kernel(e_ref, lhs_ref, rel_ref, out_hbm, buf, sems):
    i = pl.program_id(0)
    nsteps = pl.num_programs(0)
    q16 = (lhs_ref[...] * rel_ref[...]).astype(jnp.bfloat16)

    for j in range(_K):
        @pl.when(i > 0)
        def _(j=j):
            pltpu.make_async_copy(
                buf.at[j],
                out_hbm.at[pl.ds(((i - 1) * _K + j) * _BE, _BE), :],
                sems.at[j]).wait()

        acc = jax.lax.dot_general(
            e_ref[:, j * _BE:(j + 1) * _BE].astype(jnp.bfloat16), q16,
            dimension_numbers=(((0,), (1,)), ((), ())),
            preferred_element_type=jnp.float32,
        )
        buf[j] = acc
        if j < _K - 1:
            pltpu.make_async_copy(
                buf.at[j],
                out_hbm.at[pl.ds((i * _K + j) * _BE, _BE), :],
                sems.at[j]).start()
        else:
            @pl.when(i < nsteps - 1)
            def _():
                pltpu.make_async_copy(
                    buf.at[j],
                    out_hbm.at[pl.ds((i * _K + j) * _BE, _BE), :],
                    sems.at[j]).start()

            @pl.when(i == nsteps - 1)
            def _():
                pltpu.make_async_copy(
                    buf.at[j, pl.ds(0, _TAIL), :],
                    out_hbm.at[pl.ds((i * _K + j) * _BE, _TAIL), :],
                    sems.at[j]).start()

    @pl.when(i == nsteps - 1)
    def _():
        for j in range(_K - 1):
            pltpu.make_async_copy(
                buf.at[j],
                out_hbm.at[pl.ds((i * _K + j) * _BE, _BE), :],
                sems.at[j]).wait()
        pltpu.make_async_copy(
            buf.at[_K - 1, pl.ds(0, _TAIL), :],
            out_hbm.at[pl.ds((i * _K + _K - 1) * _BE, _TAIL), :],
            sems.at[_K - 1]).wait()


def _tc_scores_t(lhs, rel, entity_emb):
    e_pad = jnp.pad(entity_emb.T, ((0, 0), (0, _NP - _N)))
    grid = (_NP // (_K * _BE),)
    return pl.pallas_call(
        _score_block_kernel,
        grid=grid,
        in_specs=[
            pl.BlockSpec((_RANK, _K * _BE), lambda i: (0, i)),
            pl.BlockSpec((_B, _RANK), lambda i: (0, 0)),
            pl.BlockSpec((_B, _RANK), lambda i: (0, 0)),
        ],
        out_specs=pl.BlockSpec(memory_space=pl.ANY),
        out_shape=jax.ShapeDtypeStruct((_N, _B), jnp.float32),
        scratch_shapes=[
            pltpu.VMEM((_K, _BE, _B), jnp.float32),
            pltpu.SemaphoreType.DMA((_K,)),
        ],
        compiler_params=pltpu.CompilerParams(
            dimension_semantics=("arbitrary",),
            vmem_limit_bytes=64 * 1024 * 1024,
        ),
    )(e_pad, lhs, rel)


def kernel(queries, entity_emb, rel_emb):
    lhs_idx = queries[:, 0]
    rel_idx = queries[:, 1]
    rhs_idx = queries[:, 2]
    lhs, rel, rhs = _sc_gather_all(entity_emb, rel_emb, lhs_idx, rel_idx,
                                   rhs_idx)
    scores = _tc_scores_t(lhs, rel, entity_emb).T
    return (scores, lhs, rel, rhs)
